# Initial kernel scaffold; baseline (speedup 1.0000x reference)
import jax
import jax.numpy as jnp
from jax.experimental import pallas as pl


def kernel(input, target):
    raise NotImplementedError("write your pallas kernel here")


# speedup vs baseline: 3.1984x; 3.1984x over previous
import functools

import jax
import jax.numpy as jnp
from jax.experimental import pallas as pl
from jax.experimental.pallas import tpu as pltpu

N = 16384
D = 32
C = 128
B = 128
NB = N // B
MIN_COUNT = 10
_HI = jax.lax.Precision.HIGHEST


def _body(x_ref, t_ref, out_ref, rank_ref):
    iota_c = jax.lax.broadcasted_iota(jnp.int32, (1, C), 1)
    row = jax.lax.broadcasted_iota(jnp.int32, (B, B), 0)
    col = jax.lax.broadcasted_iota(jnp.int32, (B, B), 1)
    lstrict = (col < row).astype(jnp.float32)

    def ph1(i, counts):
        t_blk = t_ref[i]
        oh = (t_blk[:, None] == iota_c).astype(jnp.float32)
        pre = jax.lax.dot_general(lstrict, oh, (((1,), (0,)), ((), ())),
                                  precision=_HI)
        rank_mat = counts + pre
        rank_ref[i, :] = jnp.sum(rank_mat * oh, axis=1)
        return counts + jnp.sum(oh, axis=0, keepdims=True)

    counts = jax.lax.fori_loop(0, NB, ph1, jnp.zeros((1, C), jnp.float32))
    ns = jnp.floor(counts * 0.5)
    valid = counts >= float(MIN_COUNT)
    validf = valid.astype(jnp.float32)

    def ph2(i, acc):
        t_blk = t_ref[i]
        oh = (t_blk[:, None] == iota_c).astype(jnp.float32)
        ns_i = jnp.sum(ns * oh, axis=1)
        valid_i = jnp.sum(validf * oh, axis=1) > 0.5
        rank_i = rank_ref[i, :]
        w = jnp.where(valid_i & (rank_i < ns_i), 1.0, 0.0)
        xw = x_ref[i] * w[:, None]
        return acc + jax.lax.dot_general(oh, xw, (((0,), (0,)), ((), ())),
                                         precision=_HI)

    psum = jax.lax.fori_loop(0, NB, ph2, jnp.zeros((C, D), jnp.float32))

    inv_n = 1.0 / jnp.maximum(ns, 1.0)
    sn2 = jnp.sum(psum * psum, axis=1).reshape(1, C) * inv_n * inv_n
    neg_inf = jnp.float32(-jnp.inf)

    def ph3(i, carry):
        loss_sum, acc_sum, qcnt = carry
        t_blk = t_ref[i]
        x_blk = x_ref[i]
        oh_b = t_blk[:, None] == iota_c
        ohf = oh_b.astype(jnp.float32)
        xn = jnp.sum(x_blk * x_blk, axis=1, keepdims=True)
        g = jax.lax.dot_general(x_blk, psum, (((1,), (1,)), ((), ())),
                                precision=_HI)
        dist = xn - 2.0 * (g * inv_n) + sn2
        logits = jnp.where(valid, -dist, neg_inf)
        m = jnp.max(logits, axis=1, keepdims=True)
        ssum = jnp.sum(jnp.exp(logits - m), axis=1, keepdims=True)
        logit_t = jnp.sum(jnp.where(oh_b, logits, 0.0), axis=1, keepdims=True)
        logp_t = logit_t - m - jnp.log(ssum)

        ns_i = jnp.sum(ns * ohf, axis=1, keepdims=True)
        valid_i = jnp.sum(validf * ohf, axis=1, keepdims=True) > 0.5
        rank_i = rank_ref[i, :].reshape(B, 1)
        is_q = valid_i & (rank_i >= ns_i)

        loss_sum += jnp.sum(jnp.where(is_q, -logp_t, 0.0))
        qcnt += jnp.sum(jnp.where(is_q, 1.0, 0.0))
        pred = jnp.min(jnp.where(logits == m, iota_c, C), axis=1,
                       keepdims=True)
        acc_sum += jnp.sum(jnp.where((pred == t_blk[:, None]) & is_q, 1.0, 0.0))
        return loss_sum, acc_sum, qcnt

    loss_sum, acc_sum, qcnt = jax.lax.fori_loop(
        0, NB, ph3, (jnp.float32(0), jnp.float32(0), jnp.float32(0)))
    iota2 = jax.lax.broadcasted_iota(jnp.int32, (1, 2), 1)
    out_ref[...] = jnp.where(iota2 == 0, loss_sum / qcnt, acc_sum / qcnt)


@jax.jit
def kernel(input, target):
    x3 = input.reshape(NB, B, D)
    t2 = target.astype(jnp.int32).reshape(NB, B)
    out = pl.pallas_call(
        _body,
        out_shape=jax.ShapeDtypeStruct((1, 2), jnp.float32),
        scratch_shapes=[pltpu.VMEM((NB, B), jnp.float32)],
    )(x3, t2)
    return out[0, 0], out[0, 1]


# speedup vs baseline: 5.0695x; 1.5850x over previous
import functools

import jax
import jax.numpy as jnp
from jax.experimental import pallas as pl
from jax.experimental.pallas import tpu as pltpu

N = 16384
D = 32
C = 128
B = 512
NB = N // B
MIN_COUNT = 10
_HI = jax.lax.Precision.HIGHEST


def _body(x_ref, t_ref, out_ref, rank_ref):
    iota_c = jax.lax.broadcasted_iota(jnp.int32, (1, C), 1)
    row = jax.lax.broadcasted_iota(jnp.int32, (B, B), 0)
    col = jax.lax.broadcasted_iota(jnp.int32, (B, B), 1)
    lstrict = (col < row).astype(jnp.float32)

    def ph1(i, counts):
        t_blk = t_ref[i]
        oh = (t_blk[:, None] == iota_c).astype(jnp.float32)
        pre = jax.lax.dot_general(lstrict, oh, (((1,), (0,)), ((), ())),
                                  precision=_HI)
        rank_mat = counts + pre
        rank_ref[i, :] = jnp.sum(rank_mat * oh, axis=1)
        return counts + jnp.sum(oh, axis=0, keepdims=True)

    counts = jax.lax.fori_loop(0, NB, ph1, jnp.zeros((1, C), jnp.float32))
    ns = jnp.floor(counts * 0.5)
    valid = counts >= float(MIN_COUNT)
    validf = valid.astype(jnp.float32)

    def ph2(i, acc):
        t_blk = t_ref[i]
        oh = (t_blk[:, None] == iota_c).astype(jnp.float32)
        ns_i = jnp.sum(ns * oh, axis=1)
        valid_i = jnp.sum(validf * oh, axis=1) > 0.5
        rank_i = rank_ref[i, :]
        w = jnp.where(valid_i & (rank_i < ns_i), 1.0, 0.0)
        xw = x_ref[i] * w[:, None]
        return acc + jax.lax.dot_general(oh, xw, (((0,), (0,)), ((), ())),
                                         precision=_HI)

    psum = jax.lax.fori_loop(0, NB, ph2, jnp.zeros((C, D), jnp.float32))

    inv_n = 1.0 / jnp.maximum(ns, 1.0)
    sn2 = jnp.sum(psum * psum, axis=1).reshape(1, C) * inv_n * inv_n
    neg_inf = jnp.float32(-jnp.inf)

    def ph3(i, carry):
        loss_sum, acc_sum, qcnt = carry
        t_blk = t_ref[i]
        x_blk = x_ref[i]
        oh_b = t_blk[:, None] == iota_c
        ohf = oh_b.astype(jnp.float32)
        xn = jnp.sum(x_blk * x_blk, axis=1, keepdims=True)
        g = jax.lax.dot_general(x_blk, psum, (((1,), (1,)), ((), ())),
                                precision=_HI)
        dist = xn - 2.0 * (g * inv_n) + sn2
        logits = jnp.where(valid, -dist, neg_inf)
        m = jnp.max(logits, axis=1, keepdims=True)
        ssum = jnp.sum(jnp.exp(logits - m), axis=1, keepdims=True)
        logit_t = jnp.sum(jnp.where(oh_b, logits, 0.0), axis=1, keepdims=True)
        logp_t = logit_t - m - jnp.log(ssum)

        ns_i = jnp.sum(ns * ohf, axis=1, keepdims=True)
        valid_i = jnp.sum(validf * ohf, axis=1, keepdims=True) > 0.5
        rank_i = rank_ref[i, :].reshape(B, 1)
        is_q = valid_i & (rank_i >= ns_i)

        loss_sum += jnp.sum(jnp.where(is_q, -logp_t, 0.0))
        qcnt += jnp.sum(jnp.where(is_q, 1.0, 0.0))
        pred = jnp.min(jnp.where(logits == m, iota_c, C), axis=1,
                       keepdims=True)
        acc_sum += jnp.sum(jnp.where((pred == t_blk[:, None]) & is_q, 1.0, 0.0))
        return loss_sum, acc_sum, qcnt

    loss_sum, acc_sum, qcnt = jax.lax.fori_loop(
        0, NB, ph3, (jnp.float32(0), jnp.float32(0), jnp.float32(0)))
    iota2 = jax.lax.broadcasted_iota(jnp.int32, (1, 2), 1)
    out_ref[...] = jnp.where(iota2 == 0, loss_sum / qcnt, acc_sum / qcnt)


@jax.jit
def kernel(input, target):
    x3 = input.reshape(NB, B, D)
    t2 = target.astype(jnp.int32).reshape(NB, B)
    out = pl.pallas_call(
        _body,
        out_shape=jax.ShapeDtypeStruct((1, 2), jnp.float32),
        scratch_shapes=[pltpu.VMEM((NB, B), jnp.float32)],
    )(x3, t2)
    return out[0, 0], out[0, 1]


# speedup vs baseline: 7.3394x; 1.4477x over previous
import functools

import jax
import jax.numpy as jnp
from jax.experimental import pallas as pl
from jax.experimental.pallas import tpu as pltpu

N = 16384
D = 32
C = 128
B = 1024
NB = N // B
MIN_COUNT = 10
_HI = jax.lax.Precision.HIGHEST


def _body(x_ref, t_ref, out_ref, rank_ref):
    iota_c = jax.lax.broadcasted_iota(jnp.int32, (1, C), 1)
    row = jax.lax.broadcasted_iota(jnp.int32, (B, B), 0)
    col = jax.lax.broadcasted_iota(jnp.int32, (B, B), 1)
    lstrict = (col < row).astype(jnp.bfloat16)

    def ph1(i, counts):
        t_blk = t_ref[i]
        oh = (t_blk[:, None] == iota_c).astype(jnp.float32)
        pre = jax.lax.dot_general(lstrict, oh.astype(jnp.bfloat16),
                                  (((1,), (0,)), ((), ())),
                                  preferred_element_type=jnp.float32)
        rank_mat = counts + pre
        rank_ref[i, :] = jnp.sum(rank_mat * oh, axis=1)
        return counts + jnp.sum(oh, axis=0, keepdims=True)

    counts = jax.lax.fori_loop(0, NB, ph1, jnp.zeros((1, C), jnp.float32))
    ns = jnp.floor(counts * 0.5)
    valid = counts >= float(MIN_COUNT)
    validf = valid.astype(jnp.float32)

    def ph2(i, acc):
        t_blk = t_ref[i]
        oh = (t_blk[:, None] == iota_c).astype(jnp.float32)
        ns_i = jnp.sum(ns * oh, axis=1)
        valid_i = jnp.sum(validf * oh, axis=1) > 0.5
        rank_i = rank_ref[i, :]
        w = jnp.where(valid_i & (rank_i < ns_i), 1.0, 0.0)
        xw = x_ref[i] * w[:, None]
        return acc + jax.lax.dot_general(oh, xw, (((0,), (0,)), ((), ())),
                                         precision=_HI)

    psum = jax.lax.fori_loop(0, NB, ph2, jnp.zeros((C, D), jnp.float32))

    inv_n = 1.0 / jnp.maximum(ns, 1.0)
    sn2 = jnp.sum(psum * psum, axis=1).reshape(1, C) * inv_n * inv_n
    neg_inf = jnp.float32(-jnp.inf)

    def ph3(i, carry):
        loss_sum, acc_sum, qcnt = carry
        t_blk = t_ref[i]
        x_blk = x_ref[i]
        oh_b = t_blk[:, None] == iota_c
        ohf = oh_b.astype(jnp.float32)
        xn = jnp.sum(x_blk * x_blk, axis=1, keepdims=True)
        g = jax.lax.dot_general(x_blk, psum, (((1,), (1,)), ((), ())),
                                precision=_HI)
        dist = xn - 2.0 * (g * inv_n) + sn2
        logits = jnp.where(valid, -dist, neg_inf)
        m = jnp.max(logits, axis=1, keepdims=True)
        ssum = jnp.sum(jnp.exp(logits - m), axis=1, keepdims=True)
        logit_t = jnp.sum(jnp.where(oh_b, logits, 0.0), axis=1, keepdims=True)
        logp_t = logit_t - m - jnp.log(ssum)

        ns_i = jnp.sum(ns * ohf, axis=1, keepdims=True)
        valid_i = jnp.sum(validf * ohf, axis=1, keepdims=True) > 0.5
        rank_i = rank_ref[i, :].reshape(B, 1)
        is_q = valid_i & (rank_i >= ns_i)

        loss_sum += jnp.sum(jnp.where(is_q, -logp_t, 0.0))
        qcnt += jnp.sum(jnp.where(is_q, 1.0, 0.0))
        pred = jnp.min(jnp.where(logits == m, iota_c, C), axis=1,
                       keepdims=True)
        acc_sum += jnp.sum(jnp.where((pred == t_blk[:, None]) & is_q, 1.0, 0.0))
        return loss_sum, acc_sum, qcnt

    loss_sum, acc_sum, qcnt = jax.lax.fori_loop(
        0, NB, ph3, (jnp.float32(0), jnp.float32(0), jnp.float32(0)))
    iota2 = jax.lax.broadcasted_iota(jnp.int32, (1, 2), 1)
    out_ref[...] = jnp.where(iota2 == 0, loss_sum / qcnt, acc_sum / qcnt)


@jax.jit
def kernel(input, target):
    x3 = input.reshape(NB, B, D)
    t2 = target.astype(jnp.int32).reshape(NB, B)
    out = pl.pallas_call(
        _body,
        out_shape=jax.ShapeDtypeStruct((1, 2), jnp.float32),
        scratch_shapes=[pltpu.VMEM((NB, B), jnp.float32)],
    )(x3, t2)
    return out[0, 0], out[0, 1]


# speedup vs baseline: 10.8615x; 1.4799x over previous
import functools

import jax
import jax.numpy as jnp
from jax.experimental import pallas as pl
from jax.experimental.pallas import tpu as pltpu

N = 16384
D = 32
C = 128
B = 1024
NB = N // B
MIN_COUNT = 10
_HI = jax.lax.Precision.HIGHEST


def _body(x_ref, t_ref, out_ref, r1_ref):
    iota_c = jax.lax.broadcasted_iota(jnp.int32, (1, C), 1)
    row = jax.lax.broadcasted_iota(jnp.int32, (B, B), 0)
    col = jax.lax.broadcasted_iota(jnp.int32, (B, B), 1)
    lstrict = (col < row).astype(jnp.bfloat16)

    def ph1(i, counts):
        t_blk = t_ref[i]
        oh = (t_blk[:, None] == iota_c).astype(jnp.float32)
        pre = jax.lax.dot_general(lstrict, oh.astype(jnp.bfloat16),
                                  (((1,), (0,)), ((), ())),
                                  preferred_element_type=jnp.float32)
        r1_ref[i] = (counts + pre + 1.0) * oh
        return counts + jnp.sum(oh, axis=0, keepdims=True)

    counts = jax.lax.fori_loop(0, NB, ph1, jnp.zeros((1, C), jnp.float32))
    ns = jnp.floor(counts * 0.5)
    valid = counts >= float(MIN_COUNT)

    def ph2(i, acc):
        r1 = r1_ref[i]
        wsup = jnp.where((r1 >= 1.0) & (r1 <= ns) & valid, 1.0, 0.0)
        return acc + jax.lax.dot_general(wsup, x_ref[i], (((0,), (0,)), ((), ())),
                                         precision=_HI)

    psum = jax.lax.fori_loop(0, NB, ph2, jnp.zeros((C, D), jnp.float32))

    inv_n = 1.0 / jnp.maximum(ns, 1.0)
    sn2 = jnp.sum(psum * psum, axis=1).reshape(1, C) * inv_n * inv_n
    neg_inf = jnp.float32(-jnp.inf)

    def ph3(i, carry):
        loss_sum, acc_sum, qcnt = carry
        x_blk = x_ref[i]
        r1 = r1_ref[i]
        oh_b = r1 > 0.5
        xn = jnp.sum(x_blk * x_blk, axis=1, keepdims=True)
        g = jax.lax.dot_general(x_blk, psum, (((1,), (1,)), ((), ())),
                                precision=_HI)
        dist = xn - 2.0 * (g * inv_n) + sn2
        logits = jnp.where(valid, -dist, neg_inf)
        m = jnp.max(logits, axis=1, keepdims=True)
        ssum = jnp.sum(jnp.exp(logits - m), axis=1, keepdims=True)
        logit_t = jnp.sum(jnp.where(oh_b, logits, 0.0), axis=1, keepdims=True)
        logp_t = logit_t - m - jnp.log(ssum)

        is_q = jnp.sum(jnp.where(oh_b & (r1 > ns) & valid, 1.0, 0.0),
                       axis=1, keepdims=True)
        q_b = is_q > 0.5
        loss_sum += jnp.sum(jnp.where(q_b, -logp_t, 0.0))
        qcnt += jnp.sum(is_q)
        pred = jnp.min(jnp.where(logits == m, iota_c, C), axis=1,
                       keepdims=True)
        t_lane = jnp.min(jnp.where(oh_b, iota_c, C), axis=1, keepdims=True)
        acc_sum += jnp.sum(jnp.where((pred == t_lane) & q_b, 1.0, 0.0))
        return loss_sum, acc_sum, qcnt

    loss_sum, acc_sum, qcnt = jax.lax.fori_loop(
        0, NB, ph3, (jnp.float32(0), jnp.float32(0), jnp.float32(0)))
    iota2 = jax.lax.broadcasted_iota(jnp.int32, (1, 2), 1)
    out_ref[...] = jnp.where(iota2 == 0, loss_sum / qcnt, acc_sum / qcnt)


@jax.jit
def kernel(input, target):
    x3 = input.reshape(NB, B, D)
    t2 = target.astype(jnp.int32).reshape(NB, B)
    out = pl.pallas_call(
        _body,
        out_shape=jax.ShapeDtypeStruct((1, 2), jnp.float32),
        scratch_shapes=[pltpu.VMEM((NB, B, C), jnp.float32)],
    )(x3, t2)
    return out[0, 0], out[0, 1]
